# baseline (speedup 1.0000x reference)
import jax
import jax.numpy as jnp
from jax.experimental import pallas as pl
from jax.experimental.pallas import tpu as pltpu

_K = 100000
_B = 16384
_D = 64
_TAIL = _K - _B


def _push_body(mem_ref, val_ref, out_ref, sem_v, sem_m):
    cp_v = pltpu.make_async_copy(val_ref, out_ref.at[pl.ds(0, _B)], sem_v)
    cp_m = pltpu.make_async_copy(
        mem_ref.at[pl.ds(_B, _TAIL)], out_ref.at[pl.ds(_B, _TAIL)], sem_m
    )
    cp_v.start()
    cp_m.start()
    cp_v.wait()
    cp_m.wait()


def kernel(mem, value):
    return pl.pallas_call(
        _push_body,
        out_shape=jax.ShapeDtypeStruct((_K, _D), jnp.float32),
        in_specs=[
            pl.BlockSpec(memory_space=pl.ANY),
            pl.BlockSpec(memory_space=pl.ANY),
        ],
        out_specs=pl.BlockSpec(memory_space=pl.ANY),
        scratch_shapes=[pltpu.SemaphoreType.DMA, pltpu.SemaphoreType.DMA],
    )(mem, value)


# speedup vs baseline: 14.0743x; 14.0743x over previous
import jax
import jax.numpy as jnp
from jax.experimental import pallas as pl
from jax.experimental.pallas import tpu as pltpu

_K = 100000
_B = 16384
_D = 64
_BLK = 4096
_VB = _B // _BLK
_NB = pl.cdiv(_K, _BLK)


def _push_body(mem_ref, val_ref, out_ref):
    i = pl.program_id(0)

    @pl.when(i < _VB)
    def _():
        out_ref[...] = val_ref[...]

    @pl.when(i >= _VB)
    def _():
        out_ref[...] = mem_ref[...]


def kernel(mem, value):
    return pl.pallas_call(
        _push_body,
        grid=(_NB,),
        in_specs=[
            pl.BlockSpec((_BLK, _D), lambda i: (jnp.maximum(i, _VB), 0)),
            pl.BlockSpec((_BLK, _D), lambda i: (jnp.minimum(i, _VB - 1), 0)),
        ],
        out_specs=pl.BlockSpec((_BLK, _D), lambda i: (i, 0)),
        out_shape=jax.ShapeDtypeStruct((_K, _D), jnp.float32),
    )(mem, value)
